# baseline (speedup 1.0000x reference)
import functools

import jax
import jax.numpy as jnp
from jax import lax
from jax.experimental import pallas as pl
from jax.experimental.pallas import tpu as pltpu
from jax.experimental.pallas import tpu_sc as plsc

BATCH = 16384
EMB = 128
LANES = 16
NW = 32
B_PER_W = BATCH // NW
CHUNK = 128
NCHUNK = B_PER_W // CHUNK
NSL = EMB // LANES
NDIFF = 1000
DPAD = 1024
NBUF = 4
PC = 3 * EMB + LANES


def _sc_body(uid_hbm, wid_hbm, did_hbm, ut_hbm, wt_hbm, dt_hbm,
             wu_hbm, ww_hbm, wd_hbm, aux_hbm,
             out_hbm, idx_u, idx_w, idx_dd, rows_a, rows_b, rows_c, rows_d,
             rows_p, aux_v, wbuf, fold_v, par_v, acc_v, dd_v, idx_d,
             par_shared, dd_shared,
             sem_a, sem_b, sem_c, sem_e, sem_d, sem_i, sem_j, sem_k):
    c = lax.axis_index("c")
    s = lax.axis_index("s")
    w = s * 2 + c
    base = w * B_PER_W
    lane = lax.iota(jnp.int32, LANES)

    def hsum(p):
        for sh in (8, 4, 2, 1):
            p = p + p.at[lane ^ sh].get(mode="promise_in_bounds")
        return p

    masks = [lane == r for r in range(LANES)]

    def dot_rows(buf, wsl, nsl, out_ref, obase, ngroups, accum):
        def g_body(g, _):
            accv = jnp.zeros((LANES,), jnp.float32)
            for r in range(LANES):
                j = g * LANES + r
                p = buf[j, pl.ds(0, LANES)] * wsl[0]
                for k in range(1, nsl):
                    p = p + buf[j, pl.ds(k * LANES, LANES)] * wsl[k]
                accv = jnp.where(masks[r], hsum(p), accv)
            off = pl.multiple_of(obase + g * LANES, LANES)
            if accum:
                out_ref[pl.ds(off, LANES)] = out_ref[pl.ds(off, LANES)] + accv
            else:
                out_ref[pl.ds(off, LANES)] = accv
            return 0

        lax.fori_loop(0, ngroups, g_body, 0)

    h_iu = pltpu.async_copy(uid_hbm.at[pl.ds(base, B_PER_W)], idx_u, sem_i)
    h_iw = pltpu.async_copy(wid_hbm.at[pl.ds(base, B_PER_W)], idx_w, sem_j)
    h_id = pltpu.async_copy(did_hbm.at[pl.ds(base, B_PER_W)], idx_dd, sem_k)
    pltpu.sync_copy(aux_hbm, aux_v)
    h_iu.wait()
    h_iw.wait()
    h_id.wait()

    bufs = (rows_a, rows_b, rows_c, rows_d)
    sems = (sem_a, sem_b, sem_c, sem_e)
    steps = [(ut_hbm, idx_u, q) for q in range(NCHUNK)] + \
            [(wt_hbm, idx_w, q) for q in range(NCHUNK)]
    handles = [None] * NBUF

    def issue(i):
        tab, ixr, q = steps[i]
        handles[i % NBUF] = pltpu.async_copy(
            tab.at[ixr.at[pl.ds(q * CHUNK, CHUNK)]],
            bufs[i % NBUF], sems[i % NBUF])

    for i in range(NBUF - 1):
        issue(i)

    for task, w_hbm in ((0, wu_hbm), (1, wu_hbm), (2, ww_hbm), (3, ww_hbm),
                        (4, wd_hbm), (5, wd_hbm)):
        t, hs = task // 2, (task % 2) * 64

        @pl.when(s == 8 + task)
        def _(w_hbm=w_hbm, t=t, hs=hs):
            pltpu.sync_copy(w_hbm.at[pl.ds(hs, 64)], wbuf)
            ptk = [aux_v[pl.ds(t * 64 + k * LANES, LANES)] for k in range(4)]
            dot_rows(wbuf, ptk, 4, fold_v, 0, 4, accum=False)
            pltpu.sync_copy(fold_v,
                            par_shared.at[pl.ds(t * EMB + hs, 64)])

    @pl.when(s == 14)
    def _():
        cp = aux_v[pl.ds(192, LANES)] * aux_v[pl.ds(0, LANES)]
        for m in range(1, 12):
            cp = cp + (aux_v[pl.ds(192 + m * LANES, LANES)]
                       * aux_v[pl.ds(m * LANES, LANES)])
        fold_v[pl.ds(0, LANES)] = hsum(cp) + aux_v[pl.ds(384, LANES)]
        pltpu.sync_copy(fold_v.at[pl.ds(0, LANES)],
                        par_shared.at[pl.ds(3 * EMB, LANES)])

    plsc.subcore_barrier()
    pltpu.sync_copy(par_shared, par_v)

    def wslices(t):
        return [par_v[pl.ds(t * EMB + k * LANES, LANES)] for k in range(NSL)]

    @pl.when(s < 8)
    def _():
        for k in range(NSL):
            idx_d[pl.ds(k * LANES, LANES)] = jnp.minimum(
                lane + (s * CHUNK + k * LANES), NDIFF - 1)
        pltpu.async_copy(dt_hbm.at[idx_d], rows_p, sem_d).wait()
        dot_rows(rows_p, wslices(2), NSL, acc_v, 0, NSL, accum=False)
        pltpu.sync_copy(acc_v.at[pl.ds(0, CHUNK)],
                        dd_shared.at[pl.ds(s * CHUNK, CHUNK)])

    wsl_u, wsl_w = wslices(0), wslices(1)
    for i in range(len(steps)):
        if i + NBUF - 1 < len(steps):
            issue(i + NBUF - 1)
        handles[i % NBUF].wait()
        _, _, q = steps[i]
        dot_rows(bufs[i % NBUF], wsl_u if i < NCHUNK else wsl_w,
                 NSL, acc_v, q * CHUNK, NSL, accum=(i >= NCHUNK))

    plsc.subcore_barrier()

    cv = par_v[pl.ds(3 * EMB, LANES)]
    dh = [pltpu.async_copy(
        dd_shared.at[idx_dd.at[pl.ds(q * CHUNK, CHUNK)]],
        dd_v.at[pl.ds(q * CHUNK, CHUNK)], sem_d) for q in range(NCHUNK)]
    for h in dh:
        h.wait()
    for j in range(B_PER_W // LANES):
        off = j * LANES
        acc_v[pl.ds(off, LANES)] = (acc_v[pl.ds(off, LANES)]
                                    + dd_v[pl.ds(off, LANES)] + cv)

    pltpu.sync_copy(acc_v, out_hbm.at[pl.ds(base, B_PER_W)])


_gather_dot = functools.partial(
    pl.kernel,
    mesh=plsc.VectorSubcoreMesh(core_axis_name="c", subcore_axis_name="s"),
    out_type=jax.ShapeDtypeStruct((BATCH,), jnp.float32),
    scratch_types=[
        pltpu.VMEM((B_PER_W,), jnp.int32),
        pltpu.VMEM((B_PER_W,), jnp.int32),
        pltpu.VMEM((B_PER_W,), jnp.int32),
        pltpu.VMEM((CHUNK, EMB), jnp.float32),
        pltpu.VMEM((CHUNK, EMB), jnp.float32),
        pltpu.VMEM((CHUNK, EMB), jnp.float32),
        pltpu.VMEM((CHUNK, EMB), jnp.float32),
        pltpu.VMEM((CHUNK, EMB), jnp.float32),
        pltpu.VMEM((400,), jnp.float32),
        pltpu.VMEM((64, 64), jnp.float32),
        pltpu.VMEM((64,), jnp.float32),
        pltpu.VMEM((PC,), jnp.float32),
        pltpu.VMEM((B_PER_W,), jnp.float32),
        pltpu.VMEM((B_PER_W,), jnp.float32),
        pltpu.VMEM((CHUNK,), jnp.int32),
        pltpu.VMEM_SHARED((PC,), jnp.float32),
        pltpu.VMEM_SHARED((DPAD,), jnp.float32),
        pltpu.SemaphoreType.DMA,
        pltpu.SemaphoreType.DMA,
        pltpu.SemaphoreType.DMA,
        pltpu.SemaphoreType.DMA,
        pltpu.SemaphoreType.DMA,
        pltpu.SemaphoreType.DMA,
        pltpu.SemaphoreType.DMA,
        pltpu.SemaphoreType.DMA,
    ],
)(_sc_body)


def kernel(user_id, workout_id, difficulty_level_id, user_table, workout_table,
           diff_table, W_user, b_user, W_workout, b_workout, W_diff, b_diff,
           W_pred, b_pred):
    aux = jnp.concatenate([
        W_pred[:, 0], b_user, b_workout, b_diff,
        jnp.broadcast_to(b_pred, (LANES,))])
    out = _gather_dot(user_id.astype(jnp.int32), workout_id.astype(jnp.int32),
                      difficulty_level_id.astype(jnp.int32),
                      user_table, workout_table, diff_table,
                      W_user, W_workout, W_diff, aux)
    return out.reshape(BATCH, 1)


# speedup vs baseline: 1.0840x; 1.0840x over previous
import functools

import jax
import jax.numpy as jnp
from jax import lax
from jax.experimental import pallas as pl
from jax.experimental.pallas import tpu as pltpu
from jax.experimental.pallas import tpu_sc as plsc

BATCH = 16384
EMB = 128
LANES = 16
NW = 32
B_PER_W = BATCH // NW
CHUNK = 128
NCHUNK = B_PER_W // CHUNK
NSL = EMB // LANES
NDIFF = 1000
DPAD = 1024
NBUF = 4
PC = 3 * EMB + LANES


def _sc_body(uid_hbm, wid_hbm, did_hbm, ut_hbm, wt_hbm, dt_hbm,
             wu_hbm, ww_hbm, wd_hbm, aux_hbm,
             out_hbm, idx_u, idx_w, idx_dd, rows_a, rows_b, rows_c, rows_d,
             rows_p, aux_v, wbuf, fold_v, par_v, acc_v, dd_v, idx_d,
             par_shared, dd_shared,
             sem_a, sem_b, sem_c, sem_e, sem_d, sem_i, sem_j, sem_k):
    c = lax.axis_index("c")
    s = lax.axis_index("s")
    w = s * 2 + c
    base = w * B_PER_W
    lane = lax.iota(jnp.int32, LANES)

    def hsum(p):
        for sh in (8, 4, 2, 1):
            p = p + p.at[lane ^ sh].get(mode="promise_in_bounds")
        return p

    def dot_rows(buf, wsl, nsl, out_ref, obase, ngroups, accum):
        def g_body(g, _):
            def r_body(r, acc):
                j = g * LANES + r
                p = buf[j, pl.ds(0, LANES)] * wsl[0]
                for k in range(1, nsl):
                    p = p + buf[j, pl.ds(k * LANES, LANES)] * wsl[k]
                return jnp.where(lane == r, hsum(p), acc)

            accv = lax.fori_loop(0, LANES, r_body,
                                 jnp.zeros((LANES,), jnp.float32))
            off = pl.multiple_of(obase + g * LANES, LANES)
            if accum:
                out_ref[pl.ds(off, LANES)] = out_ref[pl.ds(off, LANES)] + accv
            else:
                out_ref[pl.ds(off, LANES)] = accv
            return 0

        lax.fori_loop(0, ngroups, g_body, 0)

    h_iu = pltpu.async_copy(uid_hbm.at[pl.ds(base, B_PER_W)], idx_u, sem_i)
    h_iw = pltpu.async_copy(wid_hbm.at[pl.ds(base, B_PER_W)], idx_w, sem_j)
    h_id = pltpu.async_copy(did_hbm.at[pl.ds(base, B_PER_W)], idx_dd, sem_k)
    pltpu.sync_copy(aux_hbm, aux_v)
    h_iu.wait()
    h_iw.wait()
    h_id.wait()

    bufs = (rows_a, rows_b, rows_c, rows_d)
    sems = (sem_a, sem_b, sem_c, sem_e)
    steps = [(ut_hbm, idx_u, q) for q in range(NCHUNK)] + \
            [(wt_hbm, idx_w, q) for q in range(NCHUNK)]
    handles = [None] * NBUF

    def issue(i):
        tab, ixr, q = steps[i]
        handles[i % NBUF] = pltpu.async_copy(
            tab.at[ixr.at[pl.ds(q * CHUNK, CHUNK)]],
            bufs[i % NBUF], sems[i % NBUF])

    for i in range(NBUF - 1):
        issue(i)

    task = s - 8
    t_id = lax.div(task, 2)
    hs = lax.rem(task, 2) * 64
    for k, w_hbm in ((0, wu_hbm), (1, ww_hbm), (2, wd_hbm)):
        @pl.when(jnp.logical_and(s >= 8, t_id == k))
        def _(w_hbm=w_hbm):
            pltpu.sync_copy(w_hbm.at[pl.ds(hs, 64)], wbuf)

    @pl.when(jnp.logical_and(s >= 8, s < 14))
    def _():
        ptk = [aux_v[pl.ds(t_id * 64 + k * LANES, LANES)] for k in range(4)]
        dot_rows(wbuf, ptk, 4, fold_v, 0, 4, accum=False)
        pltpu.sync_copy(fold_v, par_shared.at[pl.ds(t_id * EMB + hs, 64)])

    @pl.when(s == 14)
    def _():
        def c_body(m, cp):
            return cp + (aux_v[pl.ds(192 + m * LANES, LANES)]
                         * aux_v[pl.ds(m * LANES, LANES)])

        cp = lax.fori_loop(0, 12, c_body, jnp.zeros((LANES,), jnp.float32))
        fold_v[pl.ds(0, LANES)] = hsum(cp) + aux_v[pl.ds(384, LANES)]
        pltpu.sync_copy(fold_v.at[pl.ds(0, LANES)],
                        par_shared.at[pl.ds(3 * EMB, LANES)])

    plsc.subcore_barrier()
    pltpu.sync_copy(par_shared, par_v)

    def wslices(t):
        return [par_v[pl.ds(t * EMB + k * LANES, LANES)] for k in range(NSL)]

    @pl.when(s < 8)
    def _():
        def i_body(k, _):
            idx_d[pl.ds(k * LANES, LANES)] = jnp.minimum(
                lane + (s * CHUNK + k * LANES), NDIFF - 1)
            return 0

        lax.fori_loop(0, NSL, i_body, 0)
        pltpu.async_copy(dt_hbm.at[idx_d], rows_p, sem_d).wait()
        dot_rows(rows_p, wslices(2), NSL, acc_v, 0, NSL, accum=False)
        pltpu.sync_copy(acc_v.at[pl.ds(0, CHUNK)],
                        dd_shared.at[pl.ds(s * CHUNK, CHUNK)])

    wsl_u, wsl_w = wslices(0), wslices(1)
    for i in range(len(steps)):
        if i + NBUF - 1 < len(steps):
            issue(i + NBUF - 1)
        handles[i % NBUF].wait()
        _, _, q = steps[i]
        dot_rows(bufs[i % NBUF], wsl_u if i < NCHUNK else wsl_w,
                 NSL, acc_v, q * CHUNK, NSL, accum=(i >= NCHUNK))

    plsc.subcore_barrier()

    cv = par_v[pl.ds(3 * EMB, LANES)]
    for q in range(NCHUNK):
        pltpu.async_copy(
            dd_shared.at[idx_dd.at[pl.ds(q * CHUNK, CHUNK)]],
            dd_v.at[pl.ds(q * CHUNK, CHUNK)], sem_d).wait()

    def a_body(j, _):
        off = pl.multiple_of(j * LANES, LANES)
        acc_v[pl.ds(off, LANES)] = (acc_v[pl.ds(off, LANES)]
                                    + dd_v[pl.ds(off, LANES)] + cv)
        return 0

    lax.fori_loop(0, B_PER_W // LANES, a_body, 0)

    pltpu.sync_copy(acc_v, out_hbm.at[pl.ds(base, B_PER_W)])


_gather_dot = functools.partial(
    pl.kernel,
    mesh=plsc.VectorSubcoreMesh(core_axis_name="c", subcore_axis_name="s"),
    out_type=jax.ShapeDtypeStruct((BATCH,), jnp.float32),
    scratch_types=[
        pltpu.VMEM((B_PER_W,), jnp.int32),
        pltpu.VMEM((B_PER_W,), jnp.int32),
        pltpu.VMEM((B_PER_W,), jnp.int32),
        pltpu.VMEM((CHUNK, EMB), jnp.float32),
        pltpu.VMEM((CHUNK, EMB), jnp.float32),
        pltpu.VMEM((CHUNK, EMB), jnp.float32),
        pltpu.VMEM((CHUNK, EMB), jnp.float32),
        pltpu.VMEM((CHUNK, EMB), jnp.float32),
        pltpu.VMEM((400,), jnp.float32),
        pltpu.VMEM((64, 64), jnp.float32),
        pltpu.VMEM((64,), jnp.float32),
        pltpu.VMEM((PC,), jnp.float32),
        pltpu.VMEM((B_PER_W,), jnp.float32),
        pltpu.VMEM((B_PER_W,), jnp.float32),
        pltpu.VMEM((CHUNK,), jnp.int32),
        pltpu.VMEM_SHARED((PC,), jnp.float32),
        pltpu.VMEM_SHARED((DPAD,), jnp.float32),
        pltpu.SemaphoreType.DMA,
        pltpu.SemaphoreType.DMA,
        pltpu.SemaphoreType.DMA,
        pltpu.SemaphoreType.DMA,
        pltpu.SemaphoreType.DMA,
        pltpu.SemaphoreType.DMA,
        pltpu.SemaphoreType.DMA,
        pltpu.SemaphoreType.DMA,
    ],
)(_sc_body)


def kernel(user_id, workout_id, difficulty_level_id, user_table, workout_table,
           diff_table, W_user, b_user, W_workout, b_workout, W_diff, b_diff,
           W_pred, b_pred):
    aux = jnp.concatenate([
        W_pred[:, 0], b_user, b_workout, b_diff,
        jnp.broadcast_to(b_pred, (LANES,))])
    out = _gather_dot(user_id.astype(jnp.int32), workout_id.astype(jnp.int32),
                      difficulty_level_id.astype(jnp.int32),
                      user_table, workout_table, diff_table,
                      W_user, W_workout, W_diff, aux)
    return out.reshape(BATCH, 1)


# speedup vs baseline: 1.1113x; 1.0252x over previous
import functools

import jax
import jax.numpy as jnp
from jax import lax
from jax.experimental import pallas as pl
from jax.experimental.pallas import tpu as pltpu
from jax.experimental.pallas import tpu_sc as plsc

BATCH = 16384
EMB = 128
LANES = 16
NW = 32
B_PER_W = BATCH // NW
CHUNK = 128
NCHUNK = B_PER_W // CHUNK
NSL = EMB // LANES
NDIFF = 1000
DPAD = 1024
NBUF = 4
PC = 3 * EMB + LANES


def _sc_body(uid_hbm, wid_hbm, did_hbm, ut_hbm, wt_hbm, dt_hbm,
             w3_hbm, aux_hbm,
             out_hbm, idx_u, idx_w, idx_dd, rows_a, rows_b, rows_c, rows_d,
             rows_p, aux_v, wbuf, fold_v, par_v, acc_v, dd_v, idx_d,
             par_shared, dd_shared,
             sem_a, sem_b, sem_c, sem_e, sem_d, sem_i, sem_j, sem_k):
    c = lax.axis_index("c")
    s = lax.axis_index("s")
    w = s * 2 + c
    base = w * B_PER_W
    lane = lax.iota(jnp.int32, LANES)

    def hsum(p):
        for sh in (8, 4, 2, 1):
            p = p + p.at[lane ^ sh].get(mode="promise_in_bounds")
        return p

    def dot_rows(buf, wsl, nsl, out_ref, obase, ngroups, accum):
        def g_body(g, _):
            def r_body(r, acc):
                j = g * LANES + r
                p = buf[j, pl.ds(0, LANES)] * wsl[0]
                for k in range(1, nsl):
                    p = p + buf[j, pl.ds(k * LANES, LANES)] * wsl[k]
                return jnp.where(lane == r, hsum(p), acc)

            accv = lax.fori_loop(0, LANES, r_body,
                                 jnp.zeros((LANES,), jnp.float32))
            off = pl.multiple_of(obase + g * LANES, LANES)
            if accum:
                out_ref[pl.ds(off, LANES)] = out_ref[pl.ds(off, LANES)] + accv
            else:
                out_ref[pl.ds(off, LANES)] = accv
            return 0

        lax.fori_loop(0, ngroups, g_body, 0)

    h_iu = pltpu.async_copy(uid_hbm.at[pl.ds(base, B_PER_W)], idx_u, sem_i)
    h_iw = pltpu.async_copy(wid_hbm.at[pl.ds(base, B_PER_W)], idx_w, sem_j)
    h_id = pltpu.async_copy(did_hbm.at[pl.ds(base, B_PER_W)], idx_dd, sem_k)
    pltpu.sync_copy(aux_hbm, aux_v)
    h_iu.wait()
    h_iw.wait()
    h_id.wait()

    bufs = (rows_a, rows_b, rows_c, rows_d)
    sems = (sem_a, sem_b, sem_c, sem_e)
    steps = [(ut_hbm, idx_u, q) for q in range(NCHUNK)] + \
            [(wt_hbm, idx_w, q) for q in range(NCHUNK)]
    handles = [None] * NBUF

    def issue(i):
        tab, ixr, q = steps[i]
        handles[i % NBUF] = pltpu.async_copy(
            tab.at[ixr.at[pl.ds(q * CHUNK, CHUNK)]],
            bufs[i % NBUF], sems[i % NBUF])

    for i in range(NBUF - 1):
        issue(i)

    task = s - 8
    t_id = lax.div(task, 2)
    hs = lax.rem(task, 2) * 64

    @pl.when(jnp.logical_and(s >= 8, s < 14))
    def _():
        pltpu.sync_copy(w3_hbm.at[t_id, pl.ds(hs, 64)], wbuf)
        ptk = [aux_v[pl.ds(t_id * 64 + k * LANES, LANES)] for k in range(4)]
        dot_rows(wbuf, ptk, 4, fold_v, 0, 4, accum=False)
        pltpu.sync_copy(fold_v, par_shared.at[pl.ds(t_id * EMB + hs, 64)])

    @pl.when(s == 14)
    def _():
        def c_body(m, cp):
            return cp + (aux_v[pl.ds(192 + m * LANES, LANES)]
                         * aux_v[pl.ds(m * LANES, LANES)])

        cp = lax.fori_loop(0, 12, c_body, jnp.zeros((LANES,), jnp.float32))
        fold_v[pl.ds(0, LANES)] = hsum(cp) + aux_v[pl.ds(384, LANES)]
        pltpu.sync_copy(fold_v.at[pl.ds(0, LANES)],
                        par_shared.at[pl.ds(3 * EMB, LANES)])

    plsc.subcore_barrier()
    pltpu.sync_copy(par_shared, par_v)

    def wslices(t):
        return [par_v[pl.ds(t * EMB + k * LANES, LANES)] for k in range(NSL)]

    @pl.when(s < 8)
    def _():
        def i_body(k, _):
            idx_d[pl.ds(k * LANES, LANES)] = jnp.minimum(
                lane + (s * CHUNK + k * LANES), NDIFF - 1)
            return 0

        lax.fori_loop(0, NSL, i_body, 0)
        pltpu.async_copy(dt_hbm.at[idx_d], rows_p, sem_d).wait()
        dot_rows(rows_p, wslices(2), NSL, acc_v, 0, NSL, accum=False)
        pltpu.sync_copy(acc_v.at[pl.ds(0, CHUNK)],
                        dd_shared.at[pl.ds(s * CHUNK, CHUNK)])

    wsl_u, wsl_w = wslices(0), wslices(1)
    for i in range(len(steps)):
        if i + NBUF - 1 < len(steps):
            issue(i + NBUF - 1)
        handles[i % NBUF].wait()
        _, _, q = steps[i]
        dot_rows(bufs[i % NBUF], wsl_u if i < NCHUNK else wsl_w,
                 NSL, acc_v, q * CHUNK, NSL, accum=(i >= NCHUNK))

    plsc.subcore_barrier()

    cv = par_v[pl.ds(3 * EMB, LANES)]
    for q in range(NCHUNK):
        pltpu.async_copy(
            dd_shared.at[idx_dd.at[pl.ds(q * CHUNK, CHUNK)]],
            dd_v.at[pl.ds(q * CHUNK, CHUNK)], sem_d).wait()

    def a_body(j, _):
        off = pl.multiple_of(j * LANES, LANES)
        acc_v[pl.ds(off, LANES)] = (acc_v[pl.ds(off, LANES)]
                                    + dd_v[pl.ds(off, LANES)] + cv)
        return 0

    lax.fori_loop(0, B_PER_W // LANES, a_body, 0)

    pltpu.sync_copy(acc_v, out_hbm.at[pl.ds(base, B_PER_W)])


_gather_dot = functools.partial(
    pl.kernel,
    mesh=plsc.VectorSubcoreMesh(core_axis_name="c", subcore_axis_name="s"),
    out_type=jax.ShapeDtypeStruct((BATCH,), jnp.float32),
    scratch_types=[
        pltpu.VMEM((B_PER_W,), jnp.int32),
        pltpu.VMEM((B_PER_W,), jnp.int32),
        pltpu.VMEM((B_PER_W,), jnp.int32),
        pltpu.VMEM((CHUNK, EMB), jnp.float32),
        pltpu.VMEM((CHUNK, EMB), jnp.float32),
        pltpu.VMEM((CHUNK, EMB), jnp.float32),
        pltpu.VMEM((CHUNK, EMB), jnp.float32),
        pltpu.VMEM((CHUNK, EMB), jnp.float32),
        pltpu.VMEM((400,), jnp.float32),
        pltpu.VMEM((64, 64), jnp.float32),
        pltpu.VMEM((64,), jnp.float32),
        pltpu.VMEM((PC,), jnp.float32),
        pltpu.VMEM((B_PER_W,), jnp.float32),
        pltpu.VMEM((B_PER_W,), jnp.float32),
        pltpu.VMEM((CHUNK,), jnp.int32),
        pltpu.VMEM_SHARED((PC,), jnp.float32),
        pltpu.VMEM_SHARED((DPAD,), jnp.float32),
        pltpu.SemaphoreType.DMA,
        pltpu.SemaphoreType.DMA,
        pltpu.SemaphoreType.DMA,
        pltpu.SemaphoreType.DMA,
        pltpu.SemaphoreType.DMA,
        pltpu.SemaphoreType.DMA,
        pltpu.SemaphoreType.DMA,
        pltpu.SemaphoreType.DMA,
    ],
)(_sc_body)


def kernel(user_id, workout_id, difficulty_level_id, user_table, workout_table,
           diff_table, W_user, b_user, W_workout, b_workout, W_diff, b_diff,
           W_pred, b_pred):
    aux = jnp.concatenate([
        W_pred[:, 0], b_user, b_workout, b_diff,
        jnp.broadcast_to(b_pred, (LANES,))])
    w3 = jnp.stack([W_user, W_workout, W_diff])
    out = _gather_dot(user_id.astype(jnp.int32), workout_id.astype(jnp.int32),
                      difficulty_level_id.astype(jnp.int32),
                      user_table, workout_table, diff_table, w3, aux)
    return out.reshape(BATCH, 1)


# speedup vs baseline: 1.1610x; 1.0447x over previous
import functools

import jax
import jax.numpy as jnp
from jax import lax
from jax.experimental import pallas as pl
from jax.experimental.pallas import tpu as pltpu
from jax.experimental.pallas import tpu_sc as plsc

BATCH = 16384
EMB = 128
LANES = 16
NW = 32
B_PER_W = BATCH // NW
CHUNK = 128
NCHUNK = B_PER_W // CHUNK
NSL = EMB // LANES
NDIFF = 1000
DPAD = 1024
NBUF = 4
PC = 3 * EMB + LANES


def _sc_body(uid_hbm, wid_hbm, did_hbm, ut_hbm, wt_hbm, dt_hbm,
             big_hbm,
             out_hbm, idx_u, idx_w, idx_dd, rows_a, rows_b, rows_c, rows_d,
             rows_p, aux_v, wbuf, fold_v, par_v, acc_v, dd_v, idx_d,
             par_shared, dd_shared,
             sem_a, sem_b, sem_c, sem_e, sem_d, sem_i, sem_j, sem_k):
    c = lax.axis_index("c")
    s = lax.axis_index("s")
    w = s * 2 + c
    base = w * B_PER_W
    lane = lax.iota(jnp.int32, LANES)

    def hsum(p):
        for sh in (8, 4, 2, 1):
            p = p + p.at[lane ^ sh].get(mode="promise_in_bounds")
        return p

    def dot_rows(buf, wsl, nsl, out_ref, obase, ngroups, accum, flat=False):
        def g_body(g, _):
            def r_body(r, acc):
                j = g * LANES + r
                if flat:
                    def ld(k):
                        return buf[pl.ds(j * (LANES * nsl) + k * LANES, LANES)]
                else:
                    def ld(k):
                        return buf[j, pl.ds(k * LANES, LANES)]
                terms = [ld(k) * wsl[k] for k in range(nsl)]
                while len(terms) > 1:
                    terms = [terms[i] + terms[i + 1]
                             for i in range(0, len(terms) - 1, 2)] +                             (terms[-1:] if len(terms) % 2 else [])
                return jnp.where(lane == r, hsum(terms[0]), acc)

            accv = lax.fori_loop(0, LANES, r_body,
                                 jnp.zeros((LANES,), jnp.float32))
            off = pl.multiple_of(obase + g * LANES, LANES)
            if accum:
                out_ref[pl.ds(off, LANES)] = out_ref[pl.ds(off, LANES)] + accv
            else:
                out_ref[pl.ds(off, LANES)] = accv
            return 0

        lax.fori_loop(0, ngroups, g_body, 0)

    h_iu = pltpu.async_copy(uid_hbm.at[pl.ds(base, B_PER_W)], idx_u, sem_i)
    h_iw = pltpu.async_copy(wid_hbm.at[pl.ds(base, B_PER_W)], idx_w, sem_j)
    h_id = pltpu.async_copy(did_hbm.at[pl.ds(base, B_PER_W)], idx_dd, sem_k)
    pltpu.sync_copy(big_hbm.at[pl.ds(3 * EMB * 64, 400)], aux_v)
    h_iu.wait()
    h_iw.wait()
    h_id.wait()

    bufs = (rows_a, rows_b, rows_c, rows_d)
    sems = (sem_a, sem_b, sem_c, sem_e)
    steps = [(ut_hbm, idx_u, q) for q in range(NCHUNK)] + \
            [(wt_hbm, idx_w, q) for q in range(NCHUNK)]
    handles = [None] * NBUF

    def issue(i):
        tab, ixr, q = steps[i]
        handles[i % NBUF] = pltpu.async_copy(
            tab.at[ixr.at[pl.ds(q * CHUNK, CHUNK)]],
            bufs[i % NBUF], sems[i % NBUF])

    for i in range(NBUF - 1):
        issue(i)

    task = s - 8
    t_id = lax.div(task, 2)
    hs = lax.rem(task, 2) * 64

    @pl.when(jnp.logical_and(s >= 8, s < 14))
    def _():
        pltpu.sync_copy(
            big_hbm.at[pl.ds((t_id * EMB + hs) * 64, 64 * 64)], wbuf)
        ptk = [aux_v[pl.ds(t_id * 64 + k * LANES, LANES)] for k in range(4)]
        dot_rows(wbuf, ptk, 4, fold_v, 0, 4, accum=False, flat=True)
        pltpu.sync_copy(fold_v, par_shared.at[pl.ds(t_id * EMB + hs, 64)])

    @pl.when(s == 14)
    def _():
        def c_body(m, cp):
            return cp + (aux_v[pl.ds(192 + m * LANES, LANES)]
                         * aux_v[pl.ds(m * LANES, LANES)])

        cp = lax.fori_loop(0, 12, c_body, jnp.zeros((LANES,), jnp.float32))
        fold_v[pl.ds(0, LANES)] = hsum(cp) + aux_v[pl.ds(384, LANES)]
        pltpu.sync_copy(fold_v.at[pl.ds(0, LANES)],
                        par_shared.at[pl.ds(3 * EMB, LANES)])

    plsc.subcore_barrier()
    pltpu.sync_copy(par_shared, par_v)

    def wslices(t):
        return [par_v[pl.ds(t * EMB + k * LANES, LANES)] for k in range(NSL)]

    @pl.when(s < 8)
    def _():
        def i_body(k, _):
            idx_d[pl.ds(k * LANES, LANES)] = jnp.minimum(
                lane + (s * CHUNK + k * LANES), NDIFF - 1)
            return 0

        lax.fori_loop(0, NSL, i_body, 0)
        pltpu.async_copy(dt_hbm.at[idx_d], rows_p, sem_d).wait()
        dot_rows(rows_p, wslices(2), NSL, acc_v, 0, NSL, accum=False)
        pltpu.sync_copy(acc_v.at[pl.ds(0, CHUNK)],
                        dd_shared.at[pl.ds(s * CHUNK, CHUNK)])

    wsl_u, wsl_w = wslices(0), wslices(1)
    for i in range(len(steps)):
        if i + NBUF - 1 < len(steps):
            issue(i + NBUF - 1)
        handles[i % NBUF].wait()
        _, _, q = steps[i]
        dot_rows(bufs[i % NBUF], wsl_u if i < NCHUNK else wsl_w,
                 NSL, acc_v, q * CHUNK, NSL, accum=(i >= NCHUNK))

    plsc.subcore_barrier()

    cv = par_v[pl.ds(3 * EMB, LANES)]
    for q in range(NCHUNK):
        pltpu.async_copy(
            dd_shared.at[idx_dd.at[pl.ds(q * CHUNK, CHUNK)]],
            dd_v.at[pl.ds(q * CHUNK, CHUNK)], sem_d).wait()

    def a_body(j, _):
        off = pl.multiple_of(j * LANES, LANES)
        acc_v[pl.ds(off, LANES)] = (acc_v[pl.ds(off, LANES)]
                                    + dd_v[pl.ds(off, LANES)] + cv)
        return 0

    lax.fori_loop(0, B_PER_W // LANES, a_body, 0)

    pltpu.sync_copy(acc_v, out_hbm.at[pl.ds(base, B_PER_W)])


_gather_dot = functools.partial(
    pl.kernel,
    mesh=plsc.VectorSubcoreMesh(core_axis_name="c", subcore_axis_name="s"),
    out_type=jax.ShapeDtypeStruct((BATCH,), jnp.float32),
    scratch_types=[
        pltpu.VMEM((B_PER_W,), jnp.int32),
        pltpu.VMEM((B_PER_W,), jnp.int32),
        pltpu.VMEM((B_PER_W,), jnp.int32),
        pltpu.VMEM((CHUNK, EMB), jnp.float32),
        pltpu.VMEM((CHUNK, EMB), jnp.float32),
        pltpu.VMEM((CHUNK, EMB), jnp.float32),
        pltpu.VMEM((CHUNK, EMB), jnp.float32),
        pltpu.VMEM((CHUNK, EMB), jnp.float32),
        pltpu.VMEM((400,), jnp.float32),
        pltpu.VMEM((64 * 64,), jnp.float32),
        pltpu.VMEM((64,), jnp.float32),
        pltpu.VMEM((PC,), jnp.float32),
        pltpu.VMEM((B_PER_W,), jnp.float32),
        pltpu.VMEM((B_PER_W,), jnp.float32),
        pltpu.VMEM((CHUNK,), jnp.int32),
        pltpu.VMEM_SHARED((PC,), jnp.float32),
        pltpu.VMEM_SHARED((DPAD,), jnp.float32),
        pltpu.SemaphoreType.DMA,
        pltpu.SemaphoreType.DMA,
        pltpu.SemaphoreType.DMA,
        pltpu.SemaphoreType.DMA,
        pltpu.SemaphoreType.DMA,
        pltpu.SemaphoreType.DMA,
        pltpu.SemaphoreType.DMA,
        pltpu.SemaphoreType.DMA,
    ],
)(_sc_body)


def kernel(user_id, workout_id, difficulty_level_id, user_table, workout_table,
           diff_table, W_user, b_user, W_workout, b_workout, W_diff, b_diff,
           W_pred, b_pred):
    big = jnp.concatenate([
        W_user.reshape(-1), W_workout.reshape(-1), W_diff.reshape(-1),
        W_pred[:, 0], b_user, b_workout, b_diff,
        jnp.broadcast_to(b_pred, (LANES,))])
    out = _gather_dot(user_id.astype(jnp.int32), workout_id.astype(jnp.int32),
                      difficulty_level_id.astype(jnp.int32),
                      user_table, workout_table, diff_table, big)
    return out.reshape(BATCH, 1)


# speedup vs baseline: 1.1934x; 1.0279x over previous
import functools

import jax
import jax.numpy as jnp
from jax import lax
from jax.experimental import pallas as pl
from jax.experimental.pallas import tpu as pltpu
from jax.experimental.pallas import tpu_sc as plsc

BATCH = 16384
EMB = 128
LANES = 16
NW = 32
B_PER_W = BATCH // NW
CHUNK = 128
NCHUNK = B_PER_W // CHUNK
NSL = EMB // LANES
NDIFF = 1000
DPAD = 1024
NBUF = 4
PC = 3 * EMB + LANES


def _sc_body(uid_hbm, wid_hbm, did_hbm, ut_hbm, wt_hbm, dt_hbm,
             big_hbm,
             out_hbm, idx_u, idx_w, idx_dd, rows_a, rows_b, rows_c, rows_d,
             rows_p, aux_v, wbuf, fold_v, par_v, acc_v, dd_v, idx_d,
             par_shared, dd_shared,
             sem_a, sem_b, sem_c, sem_e, sem_d, sem_i, sem_j, sem_k):
    c = lax.axis_index("c")
    s = lax.axis_index("s")
    w = s * 2 + c
    base = w * B_PER_W
    lane = lax.iota(jnp.int32, LANES)

    def hsum(p):
        for sh in (8, 4, 2, 1):
            p = p + p.at[lane ^ sh].get(mode="promise_in_bounds")
        return p

    def dot_rows(buf, wsl, nsl, out_ref, obase, ngroups, accum, flat=False):
        def g_body(g, _):
            def r_body(r, acc):
                j = g * LANES + r
                if flat:
                    def ld(k):
                        return buf[pl.ds(j * (LANES * nsl) + k * LANES, LANES)]
                else:
                    def ld(k):
                        return buf[j, pl.ds(k * LANES, LANES)]
                terms = [ld(k) * wsl[k] for k in range(nsl)]
                while len(terms) > 1:
                    terms = [terms[i] + terms[i + 1]
                             for i in range(0, len(terms) - 1, 2)] +                             (terms[-1:] if len(terms) % 2 else [])
                return jnp.where(lane == r, hsum(terms[0]), acc)

            accv = lax.fori_loop(0, LANES, r_body,
                                 jnp.zeros((LANES,), jnp.float32))
            off = pl.multiple_of(obase + g * LANES, LANES)
            if accum:
                out_ref[pl.ds(off, LANES)] = out_ref[pl.ds(off, LANES)] + accv
            else:
                out_ref[pl.ds(off, LANES)] = accv
            return 0

        lax.fori_loop(0, ngroups, g_body, 0)

    h_iu = pltpu.async_copy(uid_hbm.at[pl.ds(base, B_PER_W)], idx_u, sem_i)
    h_iw = pltpu.async_copy(wid_hbm.at[pl.ds(base, B_PER_W)], idx_w, sem_j)
    h_id = pltpu.async_copy(did_hbm.at[pl.ds(base, B_PER_W)], idx_dd, sem_k)
    pltpu.sync_copy(big_hbm.at[pl.ds(3 * EMB * 64, 400)], aux_v)
    h_iu.wait()
    h_iw.wait()
    h_id.wait()

    bufs = (rows_a, rows_b, rows_c, rows_d)
    sems = (sem_a, sem_b, sem_c, sem_e)
    steps = [(ut_hbm, idx_u, q) for q in range(NCHUNK)] + \
            [(wt_hbm, idx_w, q) for q in range(NCHUNK)]
    handles = [None] * NBUF

    def issue(i):
        tab, ixr, q = steps[i]
        handles[i % NBUF] = pltpu.async_copy(
            tab.at[ixr.at[pl.ds(q * CHUNK, CHUNK)]],
            bufs[i % NBUF], sems[i % NBUF])

    for i in range(NBUF - 1):
        issue(i)

    task = s - 8
    t_id = lax.div(task, 2)
    hs = lax.rem(task, 2) * 64

    @pl.when(jnp.logical_and(s >= 8, s < 14))
    def _():
        pltpu.sync_copy(
            big_hbm.at[pl.ds((t_id * EMB + hs) * 64, 64 * 64)], wbuf)
        ptk = [aux_v[pl.ds(t_id * 64 + k * LANES, LANES)] for k in range(4)]
        dot_rows(wbuf, ptk, 4, fold_v, 0, 4, accum=False, flat=True)
        pltpu.sync_copy(fold_v, par_shared.at[pl.ds(t_id * EMB + hs, 64)])

    @pl.when(s == 14)
    def _():
        def c_body(m, cp):
            return cp + (aux_v[pl.ds(192 + m * LANES, LANES)]
                         * aux_v[pl.ds(m * LANES, LANES)])

        cp = lax.fori_loop(0, 12, c_body, jnp.zeros((LANES,), jnp.float32))
        fold_v[pl.ds(0, LANES)] = hsum(cp) + aux_v[pl.ds(384, LANES)]
        pltpu.sync_copy(fold_v.at[pl.ds(0, LANES)],
                        par_shared.at[pl.ds(3 * EMB, LANES)])

    plsc.subcore_barrier()
    pltpu.sync_copy(par_shared, par_v)

    def wslices(t):
        return [par_v[pl.ds(t * EMB + k * LANES, LANES)] for k in range(NSL)]

    def i_body(k, _):
        idx_d[pl.ds(k * LANES, LANES)] = jnp.minimum(
            lane + (s * 64 + k * LANES), NDIFF - 1)
        return 0

    lax.fori_loop(0, 4, i_body, 0)
    pltpu.async_copy(dt_hbm.at[idx_d.at[pl.ds(0, 64)]], rows_p, sem_d).wait()
    dot_rows(rows_p, wslices(2), NSL, acc_v, 0, 4, accum=False)
    pltpu.sync_copy(acc_v.at[pl.ds(0, 64)],
                    dd_shared.at[pl.ds(s * 64, 64)])

    wsl_u, wsl_w = wslices(0), wslices(1)
    for i in range(len(steps)):
        if i + NBUF - 1 < len(steps):
            issue(i + NBUF - 1)
        handles[i % NBUF].wait()
        _, _, q = steps[i]
        dot_rows(bufs[i % NBUF], wsl_u if i < NCHUNK else wsl_w,
                 NSL, acc_v, q * CHUNK, NSL, accum=(i >= NCHUNK))

    plsc.subcore_barrier()

    cv = par_v[pl.ds(3 * EMB, LANES)]
    for q in range(NCHUNK):
        pltpu.async_copy(
            dd_shared.at[idx_dd.at[pl.ds(q * CHUNK, CHUNK)]],
            dd_v.at[pl.ds(q * CHUNK, CHUNK)], sem_d).wait()

    def a_body(j, _):
        off = pl.multiple_of(j * LANES, LANES)
        acc_v[pl.ds(off, LANES)] = (acc_v[pl.ds(off, LANES)]
                                    + dd_v[pl.ds(off, LANES)] + cv)
        return 0

    lax.fori_loop(0, B_PER_W // LANES, a_body, 0)

    pltpu.sync_copy(acc_v, out_hbm.at[pl.ds(base, B_PER_W)])


_gather_dot = functools.partial(
    pl.kernel,
    mesh=plsc.VectorSubcoreMesh(core_axis_name="c", subcore_axis_name="s"),
    out_type=jax.ShapeDtypeStruct((BATCH,), jnp.float32),
    scratch_types=[
        pltpu.VMEM((B_PER_W,), jnp.int32),
        pltpu.VMEM((B_PER_W,), jnp.int32),
        pltpu.VMEM((B_PER_W,), jnp.int32),
        pltpu.VMEM((CHUNK, EMB), jnp.float32),
        pltpu.VMEM((CHUNK, EMB), jnp.float32),
        pltpu.VMEM((CHUNK, EMB), jnp.float32),
        pltpu.VMEM((CHUNK, EMB), jnp.float32),
        pltpu.VMEM((64, EMB), jnp.float32),
        pltpu.VMEM((400,), jnp.float32),
        pltpu.VMEM((64 * 64,), jnp.float32),
        pltpu.VMEM((64,), jnp.float32),
        pltpu.VMEM((PC,), jnp.float32),
        pltpu.VMEM((B_PER_W,), jnp.float32),
        pltpu.VMEM((B_PER_W,), jnp.float32),
        pltpu.VMEM((CHUNK,), jnp.int32),
        pltpu.VMEM_SHARED((PC,), jnp.float32),
        pltpu.VMEM_SHARED((DPAD,), jnp.float32),
        pltpu.SemaphoreType.DMA,
        pltpu.SemaphoreType.DMA,
        pltpu.SemaphoreType.DMA,
        pltpu.SemaphoreType.DMA,
        pltpu.SemaphoreType.DMA,
        pltpu.SemaphoreType.DMA,
        pltpu.SemaphoreType.DMA,
        pltpu.SemaphoreType.DMA,
    ],
)(_sc_body)


def kernel(user_id, workout_id, difficulty_level_id, user_table, workout_table,
           diff_table, W_user, b_user, W_workout, b_workout, W_diff, b_diff,
           W_pred, b_pred):
    big = jnp.concatenate([
        W_user.reshape(-1), W_workout.reshape(-1), W_diff.reshape(-1),
        W_pred[:, 0], b_user, b_workout, b_diff,
        jnp.broadcast_to(b_pred, (LANES,))])
    out = _gather_dot(user_id.astype(jnp.int32), workout_id.astype(jnp.int32),
                      difficulty_level_id.astype(jnp.int32),
                      user_table, workout_table, diff_table, big)
    return out.reshape(BATCH, 1)


# speedup vs baseline: 1.2248x; 1.0263x over previous
import functools

import jax
import jax.numpy as jnp
from jax import lax
from jax.experimental import pallas as pl
from jax.experimental.pallas import tpu as pltpu
from jax.experimental.pallas import tpu_sc as plsc

BATCH = 16384
EMB = 128
LANES = 16
NW = 32
B_PER_W = BATCH // NW
CHUNK = 128
NCHUNK = B_PER_W // CHUNK
NSL = EMB // LANES
NDIFF = 1000
DPAD = 1024
NBUF = 4
PC = 3 * EMB + LANES


def _sc_body(uid_hbm, wid_hbm, did_hbm, ut_hbm, wt_hbm, dt_hbm,
             big_hbm,
             out_hbm, idx_u, idx_w, idx_dd, rows_a, rows_b, rows_c, rows_d,
             rows_p, aux_v, wbuf, fold_v, par_v, acc_v, dd_v, idx_d,
             par_shared, dd_shared,
             sem_a, sem_b, sem_c, sem_e, sem_d, sem_i, sem_j, sem_k):
    c = lax.axis_index("c")
    s = lax.axis_index("s")
    w = s * 2 + c
    base = w * B_PER_W
    lane = lax.iota(jnp.int32, LANES)

    def hsum(p):
        for sh in (8, 4, 2, 1):
            p = p + p.at[lane ^ sh].get(mode="promise_in_bounds")
        return p

    def dot_rows(buf, wsl, nsl, out_ref, obase, ngroups, accum, flat=False):
        def g_body(g, _):
            def r_body(r, acc):
                j = g * LANES + r
                if flat:
                    def ld(k):
                        return buf[pl.ds(j * (LANES * nsl) + k * LANES, LANES)]
                else:
                    def ld(k):
                        return buf[j, pl.ds(k * LANES, LANES)]
                terms = [ld(k) * wsl[k] for k in range(nsl)]
                while len(terms) > 1:
                    terms = [terms[i] + terms[i + 1]
                             for i in range(0, len(terms) - 1, 2)] +                             (terms[-1:] if len(terms) % 2 else [])
                return jnp.where(lane == r, hsum(terms[0]), acc)

            accv = lax.fori_loop(0, LANES, r_body,
                                 jnp.zeros((LANES,), jnp.float32))
            off = pl.multiple_of(obase + g * LANES, LANES)
            if accum:
                out_ref[pl.ds(off, LANES)] = out_ref[pl.ds(off, LANES)] + accv
            else:
                out_ref[pl.ds(off, LANES)] = accv
            return 0

        lax.fori_loop(0, ngroups, g_body, 0)

    h_iu = pltpu.async_copy(uid_hbm.at[pl.ds(base, B_PER_W)], idx_u, sem_i)
    h_iw = pltpu.async_copy(wid_hbm.at[pl.ds(base, B_PER_W)], idx_w, sem_j)
    h_id = pltpu.async_copy(did_hbm.at[pl.ds(base, B_PER_W)], idx_dd, sem_k)
    pltpu.sync_copy(big_hbm.at[pl.ds(3 * EMB * 64, 400)], aux_v)
    h_iu.wait()
    h_iw.wait()
    h_id.wait()

    bufs = (rows_a, rows_b, rows_c, rows_d)
    sems = (sem_a, sem_b, sem_c, sem_e)
    steps = [(ut_hbm, idx_u, q) for q in range(NCHUNK)] + \
            [(wt_hbm, idx_w, q) for q in range(NCHUNK)]
    handles = [None] * NBUF

    def issue(i):
        tab, ixr, q = steps[i]
        handles[i % NBUF] = pltpu.async_copy(
            tab.at[ixr.at[pl.ds(q * CHUNK, CHUNK)]],
            bufs[i % NBUF], sems[i % NBUF])

    for i in range(NBUF - 1):
        issue(i)

    task = s - 8
    t_id = lax.div(task, 2)
    hs = lax.rem(task, 2) * 64

    @pl.when(jnp.logical_and(s >= 8, s < 14))
    def _():
        pltpu.sync_copy(
            big_hbm.at[pl.ds((t_id * EMB + hs) * 64, 64 * 64)], wbuf)
        ptk = [aux_v[pl.ds(t_id * 64 + k * LANES, LANES)] for k in range(4)]
        dot_rows(wbuf, ptk, 4, fold_v, 0, 4, accum=False, flat=True)
        pltpu.sync_copy(fold_v, par_shared.at[pl.ds(t_id * EMB + hs, 64)])

    @pl.when(s == 14)
    def _():
        def c_body(m, cp):
            return cp + (aux_v[pl.ds(192 + m * LANES, LANES)]
                         * aux_v[pl.ds(m * LANES, LANES)])

        cp = lax.fori_loop(0, 12, c_body, jnp.zeros((LANES,), jnp.float32))
        fold_v[pl.ds(0, LANES)] = hsum(cp) + aux_v[pl.ds(384, LANES)]
        pltpu.sync_copy(fold_v.at[pl.ds(0, LANES)],
                        par_shared.at[pl.ds(3 * EMB, LANES)])

    plsc.subcore_barrier()
    pltpu.sync_copy(par_shared, par_v)

    def wslices(t):
        return [par_v[pl.ds(t * EMB + k * LANES, LANES)] for k in range(NSL)]

    def i_body(k, _):
        idx_d[pl.ds(k * LANES, LANES)] = jnp.minimum(
            lane + (s * 64 + k * LANES), NDIFF - 1)
        return 0

    lax.fori_loop(0, 4, i_body, 0)
    pltpu.async_copy(dt_hbm.at[idx_d.at[pl.ds(0, 64)]], rows_p, sem_d).wait()
    dot_rows(rows_p, wslices(2), NSL, acc_v, 0, 4, accum=False)
    pltpu.sync_copy(acc_v.at[pl.ds(0, 64)],
                    dd_shared.at[pl.ds(s * 64, 64)])

    plsc.subcore_barrier()
    dd_handles = [pltpu.async_copy(
        dd_shared.at[idx_dd.at[pl.ds(q * CHUNK, CHUNK)]],
        dd_v.at[pl.ds(q * CHUNK, CHUNK)], sem_d) for q in range(NCHUNK)]

    wsl_u, wsl_w = wslices(0), wslices(1)
    for i in range(len(steps)):
        if i + NBUF - 1 < len(steps):
            issue(i + NBUF - 1)
        handles[i % NBUF].wait()
        _, _, q = steps[i]
        dot_rows(bufs[i % NBUF], wsl_u if i < NCHUNK else wsl_w,
                 NSL, acc_v, q * CHUNK, NSL, accum=(i >= NCHUNK))

    cv = par_v[pl.ds(3 * EMB, LANES)]
    for h in dd_handles:
        h.wait()

    def a_body(j, _):
        off = pl.multiple_of(j * LANES, LANES)
        acc_v[pl.ds(off, LANES)] = (acc_v[pl.ds(off, LANES)]
                                    + dd_v[pl.ds(off, LANES)] + cv)
        return 0

    lax.fori_loop(0, B_PER_W // LANES, a_body, 0)

    pltpu.sync_copy(acc_v, out_hbm.at[pl.ds(base, B_PER_W)])


_gather_dot = functools.partial(
    pl.kernel,
    mesh=plsc.VectorSubcoreMesh(core_axis_name="c", subcore_axis_name="s"),
    out_type=jax.ShapeDtypeStruct((BATCH,), jnp.float32),
    scratch_types=[
        pltpu.VMEM((B_PER_W,), jnp.int32),
        pltpu.VMEM((B_PER_W,), jnp.int32),
        pltpu.VMEM((B_PER_W,), jnp.int32),
        pltpu.VMEM((CHUNK, EMB), jnp.float32),
        pltpu.VMEM((CHUNK, EMB), jnp.float32),
        pltpu.VMEM((CHUNK, EMB), jnp.float32),
        pltpu.VMEM((CHUNK, EMB), jnp.float32),
        pltpu.VMEM((64, EMB), jnp.float32),
        pltpu.VMEM((400,), jnp.float32),
        pltpu.VMEM((64 * 64,), jnp.float32),
        pltpu.VMEM((64,), jnp.float32),
        pltpu.VMEM((PC,), jnp.float32),
        pltpu.VMEM((B_PER_W,), jnp.float32),
        pltpu.VMEM((B_PER_W,), jnp.float32),
        pltpu.VMEM((CHUNK,), jnp.int32),
        pltpu.VMEM_SHARED((PC,), jnp.float32),
        pltpu.VMEM_SHARED((DPAD,), jnp.float32),
        pltpu.SemaphoreType.DMA,
        pltpu.SemaphoreType.DMA,
        pltpu.SemaphoreType.DMA,
        pltpu.SemaphoreType.DMA,
        pltpu.SemaphoreType.DMA,
        pltpu.SemaphoreType.DMA,
        pltpu.SemaphoreType.DMA,
        pltpu.SemaphoreType.DMA,
    ],
)(_sc_body)


def kernel(user_id, workout_id, difficulty_level_id, user_table, workout_table,
           diff_table, W_user, b_user, W_workout, b_workout, W_diff, b_diff,
           W_pred, b_pred):
    big = jnp.concatenate([
        W_user.reshape(-1), W_workout.reshape(-1), W_diff.reshape(-1),
        W_pred[:, 0], b_user, b_workout, b_diff,
        jnp.broadcast_to(b_pred, (LANES,))])
    out = _gather_dot(user_id.astype(jnp.int32), workout_id.astype(jnp.int32),
                      difficulty_level_id.astype(jnp.int32),
                      user_table, workout_table, diff_table, big)
    return out.reshape(BATCH, 1)
